# baseline (speedup 1.0000x reference)
import functools

import jax
import jax.numpy as jnp
from jax import lax
from jax.experimental import pallas as pl
from jax.experimental.pallas import tpu as pltpu
from jax.experimental.pallas import tpu_sc as plsc

_DEG = 16
_D = 128
_NSTEP = 8
_NW = 32
_CH = 128
_ECH = 64
_WEIGHTS = (1.0, 0.6, 0.4, 0.25, 0.1)
_NUM_Q, _NUM_KC, _NUM_STU = 100000, 1000, 50000
_BASES = (0, _NUM_Q * _DEG, _NUM_Q * _DEG + _NUM_KC * _DEG,
          2 * _NUM_Q * _DEG + _NUM_KC * _DEG)
_STEP_TAB = ((_NUM_Q, _BASES[0]), (_NUM_KC, _BASES[1]),
             (_NUM_Q, _BASES[2]), (_NUM_STU, _BASES[3])) * 2


def _make_walk_kernel(n_pos):
    ppw = n_pos // _NW
    nch = ppw // _CH
    nech = ppw // _ECH
    mesh = plsc.VectorSubcoreMesh(core_axis_name="c", subcore_axis_name="s")

    @functools.partial(
        pl.kernel,
        out_type=jax.ShapeDtypeStruct((n_pos, _D), jnp.float32),
        mesh=mesh,
        scratch_types=[
            pltpu.VMEM((5, nch, _CH), jnp.int32),
            pltpu.VMEM((nch, _CH), jnp.int32),
            pltpu.VMEM((nch, _CH), jnp.int32),
            pltpu.VMEM((nch, _NSTEP, _CH), jnp.int32),
            pltpu.VMEM((2, 4, _ECH, _D), jnp.float32),
            pltpu.VMEM((2, _ECH, _D), jnp.float32),
            pltpu.SemaphoreType.DMA,
            pltpu.SemaphoreType.DMA,
            pltpu.SemaphoreType.DMA,
            pltpu.SemaphoreType.DMA,
            pltpu.SemaphoreType.DMA,
        ],
    )
    def walk(seq_hbm, tqk_hbm, tkq_hbm, tqs_hbm, tsq_hbm, ch_hbm, eq_hbm,
             out_hbm, teven, todd, idxb, chb, rows, outb,
             sem_w, sem_g0, sem_g1, sem_o0, sem_o1, *, nc=2):
        wid = lax.axis_index("s") * nc + lax.axis_index("c")
        tabs = (tqk_hbm, tkq_hbm, tqs_hbm, tsq_hbm) * 2

        descs = []
        for c in range(nch):
            base = wid * ppw + c * _CH
            descs.append(pltpu.async_copy(
                seq_hbm.at[pl.ds(base, _CH)], teven.at[0, c], sem_w))
            descs.append(pltpu.async_copy(
                ch_hbm.at[:, pl.ds(base, _CH)], chb.at[c], sem_w))
        for d in descs:
            d.wait()

        for step in range(_NSTEP):
            k = step // 2
            even = step % 2 == 0
            n_src, _ = _STEP_TAB[step]

            @plsc.parallel_loop(0, nch)
            def _(c):
                for j in range(_CH // 16):
                    s = pl.ds(j * 16, 16)
                    cur = teven[k, c, s] if even else todd[c, s]
                    idxb[c, s] = chb[c, step, s] * n_src + cur

            descs = []
            for c in range(nch):
                dst = todd.at[c] if even else teven.at[k + 1, c]
                descs.append(pltpu.async_copy(
                    tabs[step].at[idxb.at[c]], dst, sem_w))
            for d in descs:
                d.wait()

        def fire(e):
            b = e % 2
            sem = sem_g0 if b == 0 else sem_g1
            c, half = divmod(e, 2)
            descs = [pltpu.async_copy(
                eq_hbm.at[teven.at[0, c, pl.ds(half * _ECH, _ECH)]],
                outb.at[b], sem)]
            descs += [pltpu.async_copy(
                eq_hbm.at[teven.at[kk, c, pl.ds(half * _ECH, _ECH)]],
                rows.at[b, kk - 1], sem)
                for kk in range(1, 5)]
            return descs

        g_descs = {0: fire(0)}
        o_descs = {}
        for e in range(nech):
            b = e % 2
            if e + 1 < nech:
                if e - 1 in o_descs:
                    o_descs.pop(e - 1).wait()
                g_descs[e + 1] = fire(e + 1)
            for d in g_descs.pop(e):
                d.wait()

            @plsc.parallel_loop(0, _ECH)
            def _(i):
                for j in range(_D // 16):
                    s = pl.ds(j * 16, 16)
                    for kk in range(1, 5):
                        plsc.addupdate(outb.at[b, i, s],
                                       rows[b, kk - 1, i, s] * _WEIGHTS[kk])

            pos = wid * ppw + e * _ECH
            sem = sem_o0 if b == 0 else sem_o1
            o_descs[e] = pltpu.async_copy(
                outb.at[b], out_hbm.at[pl.ds(pos, _ECH), :], sem)
        for d in o_descs.values():
            d.wait()

    return walk


def kernel(x_question, y_knowledge, seq_q, E_q, E_kc, E_stu,
           nbr_q_kc, nbr_kc_q, nbr_q_stu, nbr_stu_q, choices):
    bs, seq_len = seq_q.shape
    n_pos = bs * seq_len
    walk = _make_walk_kernel(n_pos)
    tqk = nbr_q_kc.T.reshape(-1)
    tkq = nbr_kc_q.T.reshape(-1)
    tqs = nbr_q_stu.T.reshape(-1)
    tsq = nbr_stu_q.T.reshape(-1)
    seq_t = seq_q.T.reshape(-1)
    ch_t = (choices.reshape(_NSTEP, bs, seq_len)
            .transpose(0, 2, 1).reshape(_NSTEP, n_pos))
    out2 = walk(seq_t, tqk, tkq, tqs, tsq, ch_t, E_q)
    hq = out2.reshape(seq_len, bs, _D).transpose(1, 0, 2)
    return (hq, hq)


# speedup vs baseline: 1.0573x; 1.0573x over previous
import functools

import jax
import jax.numpy as jnp
from jax import lax
from jax.experimental import pallas as pl
from jax.experimental.pallas import tpu as pltpu
from jax.experimental.pallas import tpu_sc as plsc

_DEG = 16
_D = 128
_NSTEP = 8
_NW = 32
_CH = 128
_ECH = 64
_WEIGHTS = (1.0, 0.6, 0.4, 0.25, 0.1)
_NUM_Q, _NUM_KC, _NUM_STU = 100000, 1000, 50000
_BASES = (0, _NUM_Q * _DEG, _NUM_Q * _DEG + _NUM_KC * _DEG,
          2 * _NUM_Q * _DEG + _NUM_KC * _DEG)
_STEP_TAB = ((_NUM_Q, _BASES[0]), (_NUM_KC, _BASES[1]),
             (_NUM_Q, _BASES[2]), (_NUM_STU, _BASES[3])) * 2


def _make_walk_kernel(n_pos):
    ppw = n_pos // _NW
    nch = ppw // _CH
    nech = ppw // _ECH
    mesh = plsc.VectorSubcoreMesh(core_axis_name="c", subcore_axis_name="s")

    @functools.partial(
        pl.kernel,
        out_type=jax.ShapeDtypeStruct((n_pos, _D), jnp.float32),
        mesh=mesh,
        scratch_types=[
            pltpu.VMEM((5, nch, _CH), jnp.int32),
            pltpu.VMEM((nch, _CH), jnp.int32),
            pltpu.VMEM((nch, _CH), jnp.int32),
            pltpu.VMEM((nch, _NSTEP, _CH), jnp.int32),
            pltpu.VMEM((2, 4, _ECH, _D), jnp.float32),
            pltpu.VMEM((2, _ECH, _D), jnp.float32),
            pltpu.SemaphoreType.DMA,
            pltpu.SemaphoreType.DMA,
            pltpu.SemaphoreType.DMA,
            pltpu.SemaphoreType.DMA,
            pltpu.SemaphoreType.DMA,
        ],
    )
    def walk(seq_hbm, tqk_hbm, tkq_hbm, tqs_hbm, tsq_hbm, ch_hbm, eq_hbm,
             out_hbm, teven, todd, idxb, chb, rows, outb,
             sem_w, sem_g0, sem_g1, sem_o0, sem_o1, *, nc=2):
        wid = lax.axis_index("s") * nc + lax.axis_index("c")
        tabs = (tqk_hbm, tkq_hbm, tqs_hbm, tsq_hbm) * 2

        descs = []
        for c in range(nch):
            base = wid * ppw + c * _CH
            descs.append(pltpu.async_copy(
                seq_hbm.at[pl.ds(base, _CH)], teven.at[0, c], sem_w))
            descs.append(pltpu.async_copy(
                ch_hbm.at[:, pl.ds(base, _CH)], chb.at[c], sem_w))
        for d in descs:
            d.wait()

        for step in range(_NSTEP):
            k = step // 2
            even = step % 2 == 0
            n_src, _ = _STEP_TAB[step]

            @plsc.parallel_loop(0, nch)
            def _(c):
                for j in range(_CH // 16):
                    s = pl.ds(j * 16, 16)
                    cur = teven[k, c, s] if even else todd[c, s]
                    idxb[c, s] = chb[c, step, s] * n_src + cur

            descs = []
            for c in range(nch):
                dst = todd.at[c] if even else teven.at[k + 1, c]
                descs.append(pltpu.async_copy(
                    tabs[step].at[idxb.at[c]], dst, sem_w))
            for d in descs:
                d.wait()

        def fire(e):
            b = e % 2
            sem = sem_g0 if b == 0 else sem_g1
            c, half = divmod(e, 2)
            descs = [pltpu.async_copy(
                eq_hbm.at[teven.at[0, c, pl.ds(half * _ECH, _ECH)]],
                outb.at[b], sem)]
            descs += [pltpu.async_copy(
                eq_hbm.at[teven.at[kk, c, pl.ds(half * _ECH, _ECH)]],
                rows.at[b, kk - 1], sem)
                for kk in range(1, 5)]
            return descs

        g_descs = {0: fire(0)}
        o_descs = {}
        for e in range(nech):
            b = e % 2
            if e + 1 < nech:
                if e - 1 in o_descs:
                    o_descs.pop(e - 1).wait()
                g_descs[e + 1] = fire(e + 1)
            for d in g_descs.pop(e):
                d.wait()

            @plsc.parallel_loop(0, _ECH)
            def _(i):
                for j in range(_D // 16):
                    s = pl.ds(j * 16, 16)
                    acc = rows[b, 0, i, s] * _WEIGHTS[1]
                    for kk in range(2, 5):
                        acc = acc + rows[b, kk - 1, i, s] * _WEIGHTS[kk]
                    plsc.addupdate(outb.at[b, i, s], acc)

            pos = wid * ppw + e * _ECH
            sem = sem_o0 if b == 0 else sem_o1
            o_descs[e] = pltpu.async_copy(
                outb.at[b], out_hbm.at[pl.ds(pos, _ECH), :], sem)
        for d in o_descs.values():
            d.wait()

    return walk


def kernel(x_question, y_knowledge, seq_q, E_q, E_kc, E_stu,
           nbr_q_kc, nbr_kc_q, nbr_q_stu, nbr_stu_q, choices):
    bs, seq_len = seq_q.shape
    n_pos = bs * seq_len
    walk = _make_walk_kernel(n_pos)
    tqk = nbr_q_kc.T.reshape(-1)
    tkq = nbr_kc_q.T.reshape(-1)
    tqs = nbr_q_stu.T.reshape(-1)
    tsq = nbr_stu_q.T.reshape(-1)
    seq_t = seq_q.T.reshape(-1)
    ch_t = (choices.reshape(_NSTEP, bs, seq_len)
            .transpose(0, 2, 1).reshape(_NSTEP, n_pos))
    out2 = walk(seq_t, tqk, tkq, tqs, tsq, ch_t, E_q)
    hq = out2.reshape(seq_len, bs, _D).transpose(1, 0, 2)
    return (hq, hq)


# speedup vs baseline: 1.1212x; 1.0604x over previous
import functools

import jax
import jax.numpy as jnp
from jax import lax
from jax.experimental import pallas as pl
from jax.experimental.pallas import tpu as pltpu
from jax.experimental.pallas import tpu_sc as plsc

_DEG = 16
_D = 128
_NSTEP = 8
_NW = 32
_CH = 128
_ECH = 64
_WEIGHTS = (1.0, 0.6, 0.4, 0.25, 0.1)
_NUM_Q, _NUM_KC, _NUM_STU = 100000, 1000, 50000
_BASES = (0, _NUM_Q * _DEG, _NUM_Q * _DEG + _NUM_KC * _DEG,
          2 * _NUM_Q * _DEG + _NUM_KC * _DEG)
_STEP_TAB = ((_NUM_Q, _BASES[0]), (_NUM_KC, _BASES[1]),
             (_NUM_Q, _BASES[2]), (_NUM_STU, _BASES[3])) * 2


def _make_walk_kernel(n_pos):
    ppw = n_pos // _NW
    nch = ppw // _CH
    nech = ppw // _ECH
    mesh = plsc.VectorSubcoreMesh(core_axis_name="c", subcore_axis_name="s")

    @functools.partial(
        pl.kernel,
        out_type=[jax.ShapeDtypeStruct((n_pos, _D), jnp.float32),
                  jax.ShapeDtypeStruct((n_pos, _D), jnp.float32)],
        mesh=mesh,
        scratch_types=[
            pltpu.VMEM((5, nch, _CH), jnp.int32),
            pltpu.VMEM((nch, _CH), jnp.int32),
            pltpu.VMEM((nch, _CH), jnp.int32),
            pltpu.VMEM((nch, _NSTEP, _CH), jnp.int32),
            pltpu.VMEM((2, 4, _ECH, _D), jnp.float32),
            pltpu.VMEM((2, _ECH, _D), jnp.float32),
            pltpu.SemaphoreType.DMA,
            pltpu.SemaphoreType.DMA,
            pltpu.SemaphoreType.DMA,
            pltpu.SemaphoreType.DMA,
            pltpu.SemaphoreType.DMA,
        ],
    )
    def walk(seq_hbm, tqk_hbm, tkq_hbm, tqs_hbm, tsq_hbm, ch_hbm, eq_hbm,
             out_hbm, out2_hbm, teven, todd, idxb, chb, rows, outb,
             sem_w, sem_g0, sem_g1, sem_o0, sem_o1, *, nc=2):
        wid = lax.axis_index("s") * nc + lax.axis_index("c")
        tabs = (tqk_hbm, tkq_hbm, tqs_hbm, tsq_hbm) * 2

        descs = []
        for c in range(nch):
            base = wid * ppw + c * _CH
            descs.append(pltpu.async_copy(
                seq_hbm.at[pl.ds(base, _CH)], teven.at[0, c], sem_w))
            descs.append(pltpu.async_copy(
                ch_hbm.at[:, pl.ds(base, _CH)], chb.at[c], sem_w))
        for d in descs:
            d.wait()

        for step in range(_NSTEP):
            k = step // 2
            even = step % 2 == 0
            n_src, _ = _STEP_TAB[step]

            @plsc.parallel_loop(0, nch)
            def _(c):
                for j in range(_CH // 16):
                    s = pl.ds(j * 16, 16)
                    cur = teven[k, c, s] if even else todd[c, s]
                    idxb[c, s] = chb[c, step, s] * n_src + cur

            descs = []
            for c in range(nch):
                dst = todd.at[c] if even else teven.at[k + 1, c]
                descs.append(pltpu.async_copy(
                    tabs[step].at[idxb.at[c]], dst, sem_w))
            for d in descs:
                d.wait()

        def fire(e):
            b = e % 2
            sem = sem_g0 if b == 0 else sem_g1
            c, half = divmod(e, 2)
            descs = [pltpu.async_copy(
                eq_hbm.at[teven.at[0, c, pl.ds(half * _ECH, _ECH)]],
                outb.at[b], sem)]
            descs += [pltpu.async_copy(
                eq_hbm.at[teven.at[kk, c, pl.ds(half * _ECH, _ECH)]],
                rows.at[b, kk - 1], sem)
                for kk in range(1, 5)]
            return descs

        g_descs = {0: fire(0)}
        o_descs = {}
        for e in range(nech):
            b = e % 2
            if e + 1 < nech:
                if e - 1 in o_descs:
                    for d in o_descs.pop(e - 1):
                        d.wait()
                g_descs[e + 1] = fire(e + 1)
            for d in g_descs.pop(e):
                d.wait()

            @plsc.parallel_loop(0, _ECH)
            def _(i):
                for j in range(_D // 16):
                    s = pl.ds(j * 16, 16)
                    acc = rows[b, 0, i, s] * _WEIGHTS[1]
                    for kk in range(2, 5):
                        acc = acc + rows[b, kk - 1, i, s] * _WEIGHTS[kk]
                    plsc.addupdate(outb.at[b, i, s], acc)

            pos = wid * ppw + e * _ECH
            sem = sem_o0 if b == 0 else sem_o1
            o_descs[e] = (
                pltpu.async_copy(
                    outb.at[b], out_hbm.at[pl.ds(pos, _ECH), :], sem),
                pltpu.async_copy(
                    outb.at[b], out2_hbm.at[pl.ds(pos, _ECH), :], sem),
            )
        for ds_pair in o_descs.values():
            for d in ds_pair:
                d.wait()

    return walk


def kernel(x_question, y_knowledge, seq_q, E_q, E_kc, E_stu,
           nbr_q_kc, nbr_kc_q, nbr_q_stu, nbr_stu_q, choices):
    bs, seq_len = seq_q.shape
    n_pos = bs * seq_len
    walk = _make_walk_kernel(n_pos)
    tqk = nbr_q_kc.T.reshape(-1)
    tkq = nbr_kc_q.T.reshape(-1)
    tqs = nbr_q_stu.T.reshape(-1)
    tsq = nbr_stu_q.T.reshape(-1)
    seq_t = seq_q.T.reshape(-1)
    ch_t = (choices.reshape(_NSTEP, bs, seq_len)
            .transpose(0, 2, 1).reshape(_NSTEP, n_pos))
    o1, o2 = walk(seq_t, tqk, tkq, tqs, tsq, ch_t, E_q)
    hq1 = o1.reshape(seq_len, bs, _D).transpose(1, 0, 2)
    hq2 = o2.reshape(seq_len, bs, _D).transpose(1, 0, 2)
    return (hq1, hq2)


# speedup vs baseline: 1.1307x; 1.0085x over previous
import functools

import jax
import jax.numpy as jnp
from jax import lax
from jax.experimental import pallas as pl
from jax.experimental.pallas import tpu as pltpu
from jax.experimental.pallas import tpu_sc as plsc

_DEG = 16
_D = 128
_NSTEP = 8
_NW = 32
_CH = 128
_ECH = 64
_WEIGHTS = (1.0, 0.6, 0.4, 0.25, 0.1)
_NUM_Q, _NUM_KC, _NUM_STU = 100000, 1000, 50000
_BASES = (0, _NUM_Q * _DEG, _NUM_Q * _DEG + _NUM_KC * _DEG,
          2 * _NUM_Q * _DEG + _NUM_KC * _DEG)
_STEP_TAB = ((_NUM_Q, _BASES[0]), (_NUM_KC, _BASES[1]),
             (_NUM_Q, _BASES[2]), (_NUM_STU, _BASES[3])) * 2


def _make_stage1_kernel(n_pos):
    ppw = n_pos // _NW
    nch = ppw // _CH
    mesh = plsc.VectorSubcoreMesh(core_axis_name="c", subcore_axis_name="s")

    @functools.partial(
        pl.kernel,
        out_type=jax.ShapeDtypeStruct((n_pos,), jnp.int32),
        mesh=mesh,
        scratch_types=[
            pltpu.VMEM((nch, _CH), jnp.int32),
            pltpu.VMEM((nch, _CH), jnp.int32),
            pltpu.VMEM((nch, _CH), jnp.int32),
            pltpu.VMEM((nch, 2, _CH), jnp.int32),
            pltpu.SemaphoreType.DMA,
        ],
    )
    def stage1(seq_hbm, tqk_hbm, tkq_hbm, ch_hbm, t2_hbm,
               curb, nxtb, idxb, chb, sem, *, nc=2):
        wid = lax.axis_index("s") * nc + lax.axis_index("c")
        descs = []
        for c in range(nch):
            base = wid * ppw + c * _CH
            descs.append(pltpu.async_copy(
                seq_hbm.at[pl.ds(base, _CH)], curb.at[c], sem))
            descs.append(pltpu.async_copy(
                ch_hbm.at[pl.ds(0, 2), pl.ds(base, _CH)], chb.at[c], sem))
        for d in descs:
            d.wait()
        for step in range(2):
            n_src = _STEP_TAB[step][0]
            tab = tqk_hbm if step == 0 else tkq_hbm

            @plsc.parallel_loop(0, nch)
            def _(c):
                for j in range(_CH // 16):
                    s = pl.ds(j * 16, 16)
                    cur = curb[c, s] if step == 0 else nxtb[c, s]
                    idxb[c, s] = chb[c, step, s] * n_src + cur

            descs = []
            for c in range(nch):
                dst = nxtb.at[c] if step == 0 else curb.at[c]
                descs.append(pltpu.async_copy(
                    tab.at[idxb.at[c]], dst, sem))
            for d in descs:
                d.wait()

        descs = []
        for c in range(nch):
            base = wid * ppw + c * _CH
            descs.append(pltpu.async_copy(
                curb.at[c], t2_hbm.at[pl.ds(base, _CH)], sem))
        for d in descs:
            d.wait()

    return stage1


def _make_walk_kernel(n_pos):
    ppw = n_pos // _NW
    nch = ppw // _CH
    nech = ppw // _ECH
    mesh = plsc.VectorSubcoreMesh(core_axis_name="c", subcore_axis_name="s")

    @functools.partial(
        pl.kernel,
        out_type=[jax.ShapeDtypeStruct((n_pos, _D), jnp.float32),
                  jax.ShapeDtypeStruct((n_pos, _D), jnp.float32)],
        mesh=mesh,
        scratch_types=[
            pltpu.VMEM((5, nch, _CH), jnp.int32),
            pltpu.VMEM((nch, _CH), jnp.int32),
            pltpu.VMEM((nch, _CH), jnp.int32),
            pltpu.VMEM((nch, _NSTEP, _CH), jnp.int32),
            pltpu.VMEM((2, 4, _ECH, _D), jnp.float32),
            pltpu.VMEM((2, _ECH, _D), jnp.float32),
            pltpu.SemaphoreType.DMA,
            pltpu.SemaphoreType.DMA,
            pltpu.SemaphoreType.DMA,
            pltpu.SemaphoreType.DMA,
            pltpu.SemaphoreType.DMA,
        ],
    )
    def walk(seq_hbm, t2_hbm, tqk_hbm, tkq_hbm, tqs_hbm, tsq_hbm, ch_hbm,
             eq_hbm, out_hbm, out2_hbm, teven, todd, idxb, chb, rows, outb,
             sem_w, sem_g0, sem_g1, sem_o0, sem_o1, *, nc=2):
        wid = lax.axis_index("s") * nc + lax.axis_index("c")
        tabs = (tqk_hbm, tkq_hbm, tqs_hbm, tsq_hbm) * 2

        descs = []
        for c in range(nch):
            base = wid * ppw + c * _CH
            descs.append(pltpu.async_copy(
                seq_hbm.at[pl.ds(base, _CH)], teven.at[0, c], sem_w))
            descs.append(pltpu.async_copy(
                t2_hbm.at[pl.ds(base, _CH)], teven.at[1, c], sem_w))
            descs.append(pltpu.async_copy(
                ch_hbm.at[:, pl.ds(base, _CH)], chb.at[c], sem_w))
        for d in descs:
            d.wait()

        for step in range(2, _NSTEP):
            k = step // 2
            even = step % 2 == 0
            n_src, _ = _STEP_TAB[step]

            @plsc.parallel_loop(0, nch)
            def _(c):
                for j in range(_CH // 16):
                    s = pl.ds(j * 16, 16)
                    cur = teven[k, c, s] if even else todd[c, s]
                    idxb[c, s] = chb[c, step, s] * n_src + cur

            descs = []
            for c in range(nch):
                dst = todd.at[c] if even else teven.at[k + 1, c]
                descs.append(pltpu.async_copy(
                    tabs[step].at[idxb.at[c]], dst, sem_w))
            for d in descs:
                d.wait()

        def fire(e):
            b = e % 2
            sem = sem_g0 if b == 0 else sem_g1
            c, half = divmod(e, 2)
            descs = [pltpu.async_copy(
                eq_hbm.at[teven.at[0, c, pl.ds(half * _ECH, _ECH)]],
                outb.at[b], sem)]
            descs += [pltpu.async_copy(
                eq_hbm.at[teven.at[kk, c, pl.ds(half * _ECH, _ECH)]],
                rows.at[b, kk - 1], sem)
                for kk in range(1, 5)]
            return descs

        g_descs = {0: fire(0)}
        o_descs = {}
        for e in range(nech):
            b = e % 2
            if e + 1 < nech:
                if e - 1 in o_descs:
                    for d in o_descs.pop(e - 1):
                        d.wait()
                g_descs[e + 1] = fire(e + 1)
            for d in g_descs.pop(e):
                d.wait()

            @plsc.parallel_loop(0, _ECH)
            def _(i):
                for j in range(_D // 16):
                    s = pl.ds(j * 16, 16)
                    acc = rows[b, 0, i, s] * _WEIGHTS[1]
                    for kk in range(2, 5):
                        acc = acc + rows[b, kk - 1, i, s] * _WEIGHTS[kk]
                    plsc.addupdate(outb.at[b, i, s], acc)

            pos = wid * ppw + e * _ECH
            sem = sem_o0 if b == 0 else sem_o1
            o_descs[e] = (
                pltpu.async_copy(
                    outb.at[b], out_hbm.at[pl.ds(pos, _ECH), :], sem),
                pltpu.async_copy(
                    outb.at[b], out2_hbm.at[pl.ds(pos, _ECH), :], sem),
            )
        for ds_pair in o_descs.values():
            for d in ds_pair:
                d.wait()

    return walk


def kernel(x_question, y_knowledge, seq_q, E_q, E_kc, E_stu,
           nbr_q_kc, nbr_kc_q, nbr_q_stu, nbr_stu_q, choices):
    bs, seq_len = seq_q.shape
    n_pos = bs * seq_len
    walk = _make_walk_kernel(n_pos)
    tqk = nbr_q_kc.T.reshape(-1)
    tkq = nbr_kc_q.T.reshape(-1)
    tqs = nbr_q_stu.T.reshape(-1)
    tsq = nbr_stu_q.T.reshape(-1)
    seq_t = seq_q.T.reshape(-1)
    ch_t = (choices.reshape(_NSTEP, bs, seq_len)
            .transpose(0, 2, 1).reshape(_NSTEP, n_pos))
    stage1 = _make_stage1_kernel(n_pos)
    t2 = stage1(seq_t, tqk, tkq, ch_t)
    o1, o2 = walk(seq_t, t2, tqk, tkq, tqs, tsq, ch_t, E_q)
    hq1 = o1.reshape(seq_len, bs, _D).transpose(1, 0, 2)
    hq2 = o2.reshape(seq_len, bs, _D).transpose(1, 0, 2)
    return (hq1, hq2)
